# baseline (speedup 1.0000x reference)
import functools

import jax
import jax.numpy as jnp
from jax.experimental import pallas as pl
from jax.experimental.pallas import tpu as pltpu

_LANE = 128
_TILE_B = 2048
_VMEM_LIMIT = 64 * 1024 * 1024


def _head_body(x_ref, w_ref, b_ref, o_ref):
    x = x_ref[...].astype(jnp.bfloat16)
    logits = jnp.dot(x, w_ref[...], preferred_element_type=jnp.float32)
    logits = logits + b_ref[...]
    n = o_ref.shape[-1]
    o_ref[...] = logits[:, :n].astype(o_ref.dtype)


def _pick_tile(B):
    if B <= _TILE_B:
        return B
    t = _TILE_B
    while B % t and t > 8:
        t //= 2
    return t


@jax.jit
def kernel(pooled_output, weight, bias):
    B, H = pooled_output.shape
    L = weight.shape[0]

    Lp = pl.cdiv(L, _LANE) * _LANE
    w_t = weight.T.astype(jnp.bfloat16)
    bias_p = bias
    if Lp != L:
        w_t = jnp.pad(w_t, ((0, 0), (0, Lp - L)))
        bias_p = jnp.pad(bias, (0, Lp - L))
    b2 = bias_p.reshape(1, Lp).astype(jnp.float32)

    tile_b = _pick_tile(B)
    grid = (pl.cdiv(B, tile_b),)

    return pl.pallas_call(
        _head_body,
        grid=grid,
        in_specs=[
            pl.BlockSpec((tile_b, H), lambda i: (i, 0)),
            pl.BlockSpec((H, Lp), lambda i: (0, 0)),
            pl.BlockSpec((1, Lp), lambda i: (0, 0)),
        ],
        out_specs=pl.BlockSpec((tile_b, L), lambda i: (i, 0)),
        out_shape=jax.ShapeDtypeStruct((B, L), pooled_output.dtype),
        compiler_params=pltpu.CompilerParams(
            dimension_semantics=("parallel",),
            vmem_limit_bytes=_VMEM_LIMIT),
        cost_estimate=pl.CostEstimate(
            flops=2 * B * H * Lp,
            transcendentals=0,
            bytes_accessed=B * H * 4 + H * Lp * 2 + B * L * 4),
    )(pooled_output, w_t, b2)


# speedup vs baseline: 1.0194x; 1.0194x over previous
import functools

import jax
import jax.numpy as jnp
from jax.experimental import pallas as pl
from jax.experimental.pallas import tpu as pltpu

_LANE = 128
_TILE_B = 4096
_VMEM_LIMIT = 64 * 1024 * 1024


def _head_body(x_ref, w_ref, b_ref, o_ref):
    x = x_ref[...].astype(jnp.bfloat16)
    logits = jnp.dot(x, w_ref[...], preferred_element_type=jnp.float32)
    logits = logits + b_ref[...]
    n = o_ref.shape[-1]
    o_ref[...] = logits[:, :n].astype(o_ref.dtype)


def _pick_tile(B):
    if B <= _TILE_B:
        return B
    t = _TILE_B
    while B % t and t > 8:
        t //= 2
    return t


@jax.jit
def kernel(pooled_output, weight, bias):
    B, H = pooled_output.shape
    L = weight.shape[0]

    Lp = pl.cdiv(L, _LANE) * _LANE
    w_t = weight.T.astype(jnp.bfloat16)
    bias_p = bias
    if Lp != L:
        w_t = jnp.pad(w_t, ((0, 0), (0, Lp - L)))
        bias_p = jnp.pad(bias, (0, Lp - L))
    b2 = bias_p.reshape(1, Lp).astype(jnp.float32)

    tile_b = _pick_tile(B)
    grid = (pl.cdiv(B, tile_b),)

    return pl.pallas_call(
        _head_body,
        grid=grid,
        in_specs=[
            pl.BlockSpec((tile_b, H), lambda i: (i, 0)),
            pl.BlockSpec((H, Lp), lambda i: (0, 0)),
            pl.BlockSpec((1, Lp), lambda i: (0, 0)),
        ],
        out_specs=pl.BlockSpec((tile_b, L), lambda i: (i, 0)),
        out_shape=jax.ShapeDtypeStruct((B, L), pooled_output.dtype),
        compiler_params=pltpu.CompilerParams(
            dimension_semantics=("parallel",),
            vmem_limit_bytes=_VMEM_LIMIT),
        cost_estimate=pl.CostEstimate(
            flops=2 * B * H * Lp,
            transcendentals=0,
            bytes_accessed=B * H * 4 + H * Lp * 2 + B * L * 4),
    )(pooled_output, w_t, b2)


# speedup vs baseline: 1.0632x; 1.0430x over previous
import functools

import jax
import jax.numpy as jnp
from jax.experimental import pallas as pl
from jax.experimental.pallas import tpu as pltpu

_LANE = 128
_TILE_B = 4096
_VMEM_LIMIT = 64 * 1024 * 1024


def _head_body(x_ref, w_ref, b_ref, o_ref):
    logits = jax.lax.dot_general(
        x_ref[...], w_ref[...],
        dimension_numbers=(((1,), (1,)), ((), ())),
        preferred_element_type=jnp.float32)
    n = o_ref.shape[-1]
    o_ref[...] = (logits + b_ref[...])[:, :n].astype(o_ref.dtype)


def _pick_tile(B):
    if B <= _TILE_B:
        return B
    t = _TILE_B
    while B % t and t > 8:
        t //= 2
    return t


@jax.jit
def kernel(pooled_output, weight, bias):
    B, H = pooled_output.shape
    L = weight.shape[0]

    Lp = pl.cdiv(L, _LANE) * _LANE
    w_p = weight
    bias_p = bias
    if Lp != L:
        w_p = jnp.pad(weight, ((0, Lp - L), (0, 0)))
        bias_p = jnp.pad(bias, (0, Lp - L))
    b2 = bias_p.reshape(1, Lp)

    tile_b = _pick_tile(B)
    grid = (pl.cdiv(B, tile_b),)

    return pl.pallas_call(
        _head_body,
        grid=grid,
        in_specs=[
            pl.BlockSpec((tile_b, H), lambda i: (i, 0)),
            pl.BlockSpec((Lp, H), lambda i: (0, 0)),
            pl.BlockSpec((1, Lp), lambda i: (0, 0)),
        ],
        out_specs=pl.BlockSpec((tile_b, L), lambda i: (i, 0)),
        out_shape=jax.ShapeDtypeStruct((B, L), pooled_output.dtype),
        compiler_params=pltpu.CompilerParams(
            dimension_semantics=("parallel",),
            vmem_limit_bytes=_VMEM_LIMIT),
        cost_estimate=pl.CostEstimate(
            flops=2 * B * H * Lp,
            transcendentals=0,
            bytes_accessed=B * H * 4 + Lp * H * 4 + B * L * 4),
    )(pooled_output, w_p, b2)
